# baseline (speedup 1.0000x reference)
import functools

import numpy as np
import jax
import jax.numpy as jnp
from jax import lax
from jax.experimental import pallas as pl
from jax.experimental.pallas import tpu as pltpu
from jax.experimental.pallas import tpu_sc as plsc

_B, _H, _D = 2048, 512, 32
_NW = 32
_SPW = _B // _NW
_NT = 8
_LUT = 1024


def _w_table():
    t = np.arange(_LUT, dtype=np.float64)
    t = np.minimum(t, _H)
    w = 1.0 + 0.7 * (t / (_H - 1)) ** 2.5
    w = w * (_D / _B)
    return jnp.asarray(w.astype(np.float32))


def _sc_body(pred_hbm, targ_hbm, lut_hbm, out_hbm,
             lut_v, tw, pw, acc_v, tsems, psem):
    c = lax.axis_index("c")
    s = lax.axis_index("s")
    wid = s * 2 + c
    base = wid * _SPW

    pltpu.sync_copy(lut_hbm, lut_v)

    lane = lax.iota(jnp.int32, 16)
    lane0 = (lane == 0).astype(jnp.float32)
    acc = jnp.zeros((16,), jnp.float32)

    def stage_targ(j):
        return pltpu.async_copy(
            targ_hbm.at[base + j, pl.ds(0, 8), :], tw.at[j % _NT],
            tsems.at[j % _NT])

    tpend = [stage_targ(j) for j in range(_NT)]

    for g in range(_SPW // 16):
        coeffv = jnp.zeros((16,), jnp.float32)
        t0v = jnp.zeros((16,), jnp.float32)
        pred_handles = []
        offs = []
        for k in range(16):
            j = g * 16 + k
            b = j % _NT
            tpend[b].wait()

            def cbody(ci, cnt, b=b):
                c0 = pl.multiple_of(ci * 64, 64)
                x = (tw[b, 1, pl.ds(c0, 16)] != 0.0).astype(jnp.float32)
                for u in range(1, 4):
                    cu = pl.multiple_of(c0 + u * 16, 16)
                    x = x + (tw[b, 1, pl.ds(cu, 16)] != 0.0).astype(
                        jnp.float32)
                return cnt + x

            cnt = lax.fori_loop(0, _H // 64, cbody,
                                jnp.zeros((16,), jnp.float32))
            t = jnp.sum(cnt)
            ti = t.astype(jnp.int32)
            safe = jnp.maximum(ti - 1, 0)

            sub = (lane == safe % 16).astype(jnp.float32)
            co16 = pl.multiple_of((safe // 16) * 16, 16)
            t0 = jnp.sum(tw[b, 0, pl.ds(co16, 16)] * sub)

            lsel = (lane == ti % 16).astype(jnp.float32)
            lo16 = pl.multiple_of((ti // 16) * 16, 16)
            coeff = jnp.sum(lut_v[pl.ds(lo16, 16)] * lsel)
            coeff = jnp.where(ti >= 1, coeff, 0.0)

            cb = pl.multiple_of((safe // 128) * 128, 128)
            offs.append((safe % 128 // 16) * 16)
            pred_handles.append(pltpu.async_copy(
                pred_hbm.at[base + j, pl.ds(0, 8), pl.ds(cb, 128)],
                pw.at[k], psem))

            sel = lane == k
            t0v = jnp.where(sel, jnp.full((16,), t0, jnp.float32), t0v)
            coeffv = jnp.where(sel, jnp.full((16,), coeff, jnp.float32),
                               coeffv)
            offs[-1] = (offs[-1], sub)

            if j + _NT < _SPW:
                tpend[b] = stage_targ(j + _NT)

        for h in pred_handles:
            h.wait()
        p0v = jnp.zeros((16,), jnp.float32)
        for k in range(16):
            po, sub = offs[k]
            chunk = pw[k, 0, pl.ds(pl.multiple_of(po, 16), 16)]
            p0 = jnp.sum(chunk * sub)
            p0v = jnp.where(lane == k, jnp.full((16,), p0, jnp.float32), p0v)
        d = p0v - t0v
        acc = acc + coeffv * d * d

    acc_v[pl.ds(0, 16)] = acc
    pltpu.sync_copy(acc_v, out_hbm.at[pl.ds(wid * 128, 128)])


def kernel(pred, targ, weights):
    predT = jnp.transpose(pred, (0, 2, 1))
    targT = jnp.transpose(targ, (0, 2, 1))
    lut = _w_table()

    mesh = plsc.VectorSubcoreMesh(core_axis_name="c", subcore_axis_name="s")
    run = functools.partial(
        pl.kernel,
        mesh=mesh,
        compiler_params=pltpu.CompilerParams(needs_layout_passes=False),
        out_type=jax.ShapeDtypeStruct((_NW * 128,), jnp.float32),
        scratch_types=[
            pltpu.VMEM((_LUT,), jnp.float32),
            pltpu.VMEM((_NT, 8, _H), jnp.float32),
            pltpu.VMEM((16, 8, 128), jnp.float32),
            pltpu.VMEM((128,), jnp.float32),
            pltpu.SemaphoreType.DMA((_NT,)),
            pltpu.SemaphoreType.DMA,
        ],
    )(_sc_body)

    flat = run(predT, targT, lut)
    partials = flat.reshape(_NW, 128)[:, :16]
    loss = jnp.sum(partials)
    return (loss, {"a0_loss": loss})


# speedup vs baseline: 1.2655x; 1.2655x over previous
import functools

import numpy as np
import jax
import jax.numpy as jnp
from jax import lax
from jax.experimental import pallas as pl
from jax.experimental.pallas import tpu as pltpu
from jax.experimental.pallas import tpu_sc as plsc

_B, _H, _D = 2048, 512, 32
_NW = 32
_SPW = _B // _NW
_LUT = 1024


def _w_table():
    t = np.arange(_LUT, dtype=np.float64)
    t = np.minimum(t, _H)
    w = 1.0 + 0.7 * (t / (_H - 1)) ** 2.5
    w = w * (_D / _B)
    return jnp.asarray(w.astype(np.float32))


def _sc_body(pred_hbm, targ_hbm, lut_hbm, out_hbm,
             lut_v, ib1, ib0, g1, g0, gp, acc_v, sems):
    c = lax.axis_index("c")
    s = lax.axis_index("s")
    wid = s * 2 + c
    base = wid * _SPW

    pltpu.sync_copy(lut_hbm, lut_v)

    lane = lax.iota(jnp.int32, 16)
    for g in range(_SPW // 16):
        rows = (base + g * 16 + lane) * _D
        ib0[pl.ds(g * 16, 16)] = rows
        ib1[pl.ds(g * 16, 16)] = rows + 1

    h1 = pltpu.async_copy(targ_hbm.at[ib1], g1, sems.at[0])
    h0 = pltpu.async_copy(targ_hbm.at[ib0], g0, sems.at[1])
    hp = pltpu.async_copy(pred_hbm.at[ib0], gp, sems.at[2])
    h1.wait()
    h0.wait()
    hp.wait()

    acc = jnp.zeros((16,), jnp.float32)
    for j in range(_SPW):
        def cbody(ci, cnt, j=j):
            c0 = pl.multiple_of(ci * 64, 64)
            x = (g1[j, pl.ds(c0, 16)] != 0.0).astype(jnp.float32)
            for u in range(1, 4):
                cu = pl.multiple_of(c0 + u * 16, 16)
                x = x + (g1[j, pl.ds(cu, 16)] != 0.0).astype(jnp.float32)
            return cnt + x

        cnt = lax.fori_loop(0, _H // 64, cbody, jnp.zeros((16,), jnp.float32))
        t = jnp.sum(cnt)
        ti = t.astype(jnp.int32)
        safe = jnp.maximum(ti - 1, 0)

        sub = (lane == safe % 16).astype(jnp.float32)
        co16 = pl.multiple_of((safe // 16) * 16, 16)
        t0 = jnp.sum(g0[j, pl.ds(co16, 16)] * sub)
        p0 = jnp.sum(gp[j, pl.ds(co16, 16)] * sub)

        lsel = (lane == ti % 16).astype(jnp.float32)
        lo16 = pl.multiple_of((ti // 16) * 16, 16)
        coeff = jnp.sum(lut_v[pl.ds(lo16, 16)] * lsel)
        coeff = jnp.where(ti >= 1, coeff, 0.0)

        d = p0 - t0
        val = coeff * d * d
        acc = acc + jnp.where(lane == j % 16,
                              jnp.full((16,), val, jnp.float32),
                              jnp.zeros((16,), jnp.float32))

    acc_v[pl.ds(0, 16)] = acc
    pltpu.sync_copy(acc_v, out_hbm.at[pl.ds(wid * 128, 128)])


def kernel(pred, targ, weights):
    predT = jnp.transpose(pred, (0, 2, 1)).reshape(_B * _D, _H)
    targT = jnp.transpose(targ, (0, 2, 1)).reshape(_B * _D, _H)
    lut = _w_table()

    mesh = plsc.VectorSubcoreMesh(core_axis_name="c", subcore_axis_name="s")
    run = functools.partial(
        pl.kernel,
        mesh=mesh,
        compiler_params=pltpu.CompilerParams(needs_layout_passes=False),
        out_type=jax.ShapeDtypeStruct((_NW * 128,), jnp.float32),
        scratch_types=[
            pltpu.VMEM((_LUT,), jnp.float32),
            pltpu.VMEM((_SPW,), jnp.int32),
            pltpu.VMEM((_SPW,), jnp.int32),
            pltpu.VMEM((_SPW, _H), jnp.float32),
            pltpu.VMEM((_SPW, _H), jnp.float32),
            pltpu.VMEM((_SPW, _H), jnp.float32),
            pltpu.VMEM((128,), jnp.float32),
            pltpu.SemaphoreType.DMA((3,)),
        ],
    )(_sc_body)

    flat = run(predT, targT, lut)
    partials = flat.reshape(_NW, 128)[:, :16]
    loss = jnp.sum(partials)
    return (loss, {"a0_loss": loss})


# speedup vs baseline: 1.3559x; 1.0715x over previous
import functools

import jax
import jax.numpy as jnp
from jax import lax
from jax.experimental import pallas as pl
from jax.experimental.pallas import tpu as pltpu
from jax.experimental.pallas import tpu_sc as plsc

_B, _H, _D = 2048, 512, 32
_NW = 32
_SPW = _B // _NW


def _sqrt16(x):
    i = plsc.bitcast(x, jnp.int32)
    y = plsc.bitcast(jax.lax.shift_right_logical(i, 1) + 0x1fbd1df5,
                     jnp.float32)
    for _ in range(3):
        y = 0.5 * (y + x / y)
    return y


def _sc_body(pred_hbm, targ_hbm, out_hbm,
             ib1, ib0, g1, g0, gp, acc_v, sems):
    c = lax.axis_index("c")
    s = lax.axis_index("s")
    wid = s * 2 + c
    base = wid * _SPW

    lane = lax.iota(jnp.int32, 16)
    for g in range(_SPW // 16):
        rows = (base + g * 16 + lane) * _D
        ib0[pl.ds(g * 16, 16)] = rows
        ib1[pl.ds(g * 16, 16)] = rows + 1

    h1 = pltpu.async_copy(targ_hbm.at[ib1], g1, sems.at[0])
    h0 = pltpu.async_copy(targ_hbm.at[ib0], g0, sems.at[1])
    hp = pltpu.async_copy(pred_hbm.at[ib0], gp, sems.at[2])
    h1.wait()
    h0.wait()
    hp.wait()

    acc = jnp.zeros((16,), jnp.float32)
    for g in range(_SPW // 16):
        tvec = jnp.zeros((16,), jnp.float32)
        t0v = jnp.zeros((16,), jnp.float32)
        p0v = jnp.zeros((16,), jnp.float32)
        for k in range(16):
            j = g * 16 + k

            def cbody(ci, cnt, j=j):
                c0 = pl.multiple_of(ci * 128, 128)
                x = (g1[j, pl.ds(c0, 16)] != 0.0).astype(jnp.float32)
                for u in range(1, 8):
                    cu = pl.multiple_of(c0 + u * 16, 16)
                    x = x + (g1[j, pl.ds(cu, 16)] != 0.0).astype(jnp.float32)
                return cnt + x

            cnt = lax.fori_loop(0, _H // 128, cbody,
                                jnp.zeros((16,), jnp.float32))
            t = jnp.sum(cnt)
            safe = jnp.maximum(t.astype(jnp.int32) - 1, 0)

            sub = (lane == safe % 16).astype(jnp.float32)
            co16 = pl.multiple_of((safe // 16) * 16, 16)
            t0 = jnp.sum(g0[j, pl.ds(co16, 16)] * sub)
            p0 = jnp.sum(gp[j, pl.ds(co16, 16)] * sub)

            sel = lane == k
            tvec = jnp.where(sel, jnp.full((16,), t, jnp.float32), tvec)
            t0v = jnp.where(sel, jnp.full((16,), t0, jnp.float32), t0v)
            p0v = jnp.where(sel, jnp.full((16,), p0, jnp.float32), p0v)

        xn = tvec * (1.0 / (_H - 1))
        w = 1.0 + 0.7 * (xn * xn) * _sqrt16(xn)
        coeff = jnp.where(tvec >= 1.0, w * (_D / _B),
                          jnp.zeros((16,), jnp.float32))
        d = p0v - t0v
        acc = acc + coeff * d * d

    acc_v[pl.ds(0, 16)] = acc
    pltpu.sync_copy(acc_v, out_hbm.at[pl.ds(wid * 128, 128)])


def kernel(pred, targ, weights):
    predT = jnp.transpose(pred, (0, 2, 1)).reshape(_B * _D, _H)
    targT = jnp.transpose(targ, (0, 2, 1)).reshape(_B * _D, _H)

    mesh = plsc.VectorSubcoreMesh(core_axis_name="c", subcore_axis_name="s")
    run = functools.partial(
        pl.kernel,
        mesh=mesh,
        compiler_params=pltpu.CompilerParams(needs_layout_passes=False),
        out_type=jax.ShapeDtypeStruct((_NW * 128,), jnp.float32),
        scratch_types=[
            pltpu.VMEM((_SPW,), jnp.int32),
            pltpu.VMEM((_SPW,), jnp.int32),
            pltpu.VMEM((_SPW, _H), jnp.float32),
            pltpu.VMEM((_SPW, _H), jnp.float32),
            pltpu.VMEM((_SPW, _H), jnp.float32),
            pltpu.VMEM((128,), jnp.float32),
            pltpu.SemaphoreType.DMA((3,)),
        ],
    )(_sc_body)

    flat = run(predT, targT)
    partials = flat.reshape(_NW, 128)[:, :16]
    loss = jnp.sum(partials)
    return (loss, {"a0_loss": loss})


# speedup vs baseline: 1.3684x; 1.0092x over previous
import functools

import jax
import jax.numpy as jnp
from jax import lax
from jax.experimental import pallas as pl
from jax.experimental.pallas import tpu as pltpu
from jax.experimental.pallas import tpu_sc as plsc

_B, _H, _D = 2048, 512, 32
_NW = 32
_SPW = _B // _NW


def _sqrt16(x):
    i = plsc.bitcast(x, jnp.int32)
    y = plsc.bitcast(jax.lax.shift_right_logical(i, 1) + 0x1fbd1df5,
                     jnp.float32)
    for _ in range(3):
        y = 0.5 * (y + x / y)
    return y


def _sc_body(pred_hbm, targ_hbm, out_hbm,
             ib1, ib0, g1, g0, gp, acc_v, sems):
    c = lax.axis_index("c")
    s = lax.axis_index("s")
    wid = s * 2 + c
    base = wid * _SPW

    lane = lax.iota(jnp.int32, 16)
    ngrp = _SPW // 16
    for g in range(ngrp):
        rows = (base + g * 16 + lane) * _D
        ib0[g, :] = rows
        ib1[g, :] = rows + 1

    h1 = []
    h0 = []
    hp = []
    for g in range(ngrp):
        sl = pl.ds(g * 16, 16)
        h1.append(pltpu.async_copy(targ_hbm.at[ib1.at[g]], g1.at[sl, :],
                                   sems.at[0, g]))
        h0.append(pltpu.async_copy(targ_hbm.at[ib0.at[g]], g0.at[sl, :],
                                   sems.at[1, g]))
        hp.append(pltpu.async_copy(pred_hbm.at[ib0.at[g]], gp.at[sl, :],
                                   sems.at[2, g]))

    acc = jnp.zeros((16,), jnp.float32)
    for g in range(ngrp):
        tvec = jnp.zeros((16,), jnp.float32)
        t0v = jnp.zeros((16,), jnp.float32)
        p0v = jnp.zeros((16,), jnp.float32)
        h1[g].wait()
        h0[g].wait()
        hp[g].wait()
        for k in range(16):
            j = g * 16 + k

            def cbody(ci, cnt, j=j):
                c0 = pl.multiple_of(ci * 128, 128)
                x = (g1[j, pl.ds(c0, 16)] != 0.0).astype(jnp.float32)
                for u in range(1, 8):
                    cu = pl.multiple_of(c0 + u * 16, 16)
                    x = x + (g1[j, pl.ds(cu, 16)] != 0.0).astype(jnp.float32)
                return cnt + x

            cnt = lax.fori_loop(0, _H // 128, cbody,
                                jnp.zeros((16,), jnp.float32))
            t = jnp.sum(cnt)
            safe = jnp.maximum(t.astype(jnp.int32) - 1, 0)

            sub = (lane == safe % 16).astype(jnp.float32)
            co16 = pl.multiple_of((safe // 16) * 16, 16)
            t0 = jnp.sum(g0[j, pl.ds(co16, 16)] * sub)
            p0 = jnp.sum(gp[j, pl.ds(co16, 16)] * sub)

            sel = lane == k
            tvec = jnp.where(sel, jnp.full((16,), t, jnp.float32), tvec)
            t0v = jnp.where(sel, jnp.full((16,), t0, jnp.float32), t0v)
            p0v = jnp.where(sel, jnp.full((16,), p0, jnp.float32), p0v)

        xn = tvec * (1.0 / (_H - 1))
        w = 1.0 + 0.7 * (xn * xn) * _sqrt16(xn)
        coeff = jnp.where(tvec >= 1.0, w * (_D / _B),
                          jnp.zeros((16,), jnp.float32))
        d = p0v - t0v
        acc = acc + coeff * d * d

    acc_v[pl.ds(0, 16)] = acc
    pltpu.sync_copy(acc_v, out_hbm.at[pl.ds(wid * 128, 128)])


def kernel(pred, targ, weights):
    predT = jnp.transpose(pred, (0, 2, 1)).reshape(_B * _D, _H)
    targT = jnp.transpose(targ, (0, 2, 1)).reshape(_B * _D, _H)

    mesh = plsc.VectorSubcoreMesh(core_axis_name="c", subcore_axis_name="s")
    run = functools.partial(
        pl.kernel,
        mesh=mesh,
        compiler_params=pltpu.CompilerParams(needs_layout_passes=False),
        out_type=jax.ShapeDtypeStruct((_NW * 128,), jnp.float32),
        scratch_types=[
            pltpu.VMEM((_SPW // 16, 16), jnp.int32),
            pltpu.VMEM((_SPW // 16, 16), jnp.int32),
            pltpu.VMEM((_SPW, _H), jnp.float32),
            pltpu.VMEM((_SPW, _H), jnp.float32),
            pltpu.VMEM((_SPW, _H), jnp.float32),
            pltpu.VMEM((128,), jnp.float32),
            pltpu.SemaphoreType.DMA((3, _SPW // 16)),
        ],
    )(_sc_body)

    flat = run(predT, targT)
    partials = flat.reshape(_NW, 128)[:, :16]
    loss = jnp.sum(partials)
    return (loss, {"a0_loss": loss})
